# baseline (speedup 1.0000x reference)
import math

import jax
import jax.numpy as jnp
from jax import lax
from jax.experimental import pallas as pl
from jax.experimental.pallas import tpu as pltpu
from jax.experimental.pallas import tpu_sc as plsc

_VOCAB = 1000000
_DIM = 64
_B = 4096
_S = 200
_NW = 32
_BW = _B // _NW
_TB = _B // 128
_SCALE = 1.0 / math.sqrt(_DIM)
_L = 16
_NBUF = 2
_PITCH = 129


def _embed_kernel(idst_hbm, table_hbm, out_hbm,
                  idx_v, gbufs, obufs, gsems, ssems):
    wid = lax.axis_index("s") * 2 + lax.axis_index("c")

    pltpu.sync_copy(idst_hbm.at[:, pl.ds(wid * _BW, _BW)], idx_v)

    iota = lax.iota(jnp.int32, _L)
    dvecs = [iota + c * _L for c in range(_DIM // _L)]

    def start_gather(s, b):
        pltpu.async_copy(table_hbm.at[idx_v.at[s]], gbufs[b], gsems[b])

    def transpose_scale(b):
        gbuf, obuf = gbufs[b], obufs[b]

        @plsc.parallel_loop(0, _BW, unroll=4)
        def _(bm):
            bmv = iota * 0 + bm
            for c in range(_DIM // _L):
                v = gbuf[bm, pl.ds(c * _L, _L)] * _SCALE
                plsc.store_scatter(obuf, [dvecs[c], bmv], v)

    def store_out(s, b):
        for td in range(8):
            pltpu.async_copy(
                obufs[b].at[pl.ds(td * 8, 8), pl.ds(0, 128)],
                out_hbm.at[s, td, wid], ssems[b])

    def wait_stores(s, b):
        for td in range(8):
            pltpu.make_async_copy(
                obufs[b].at[pl.ds(td * 8, 8), pl.ds(0, 128)],
                out_hbm.at[s, td, wid], ssems[b]).wait()

    def step(s, b, wait_store, more_gathers):
        pltpu.make_async_copy(table_hbm.at[idx_v.at[s]], gbufs[b],
                              gsems[b]).wait()
        if wait_store:
            wait_stores(s, b)
        transpose_scale(b)
        if more_gathers:
            start_gather(s + _NBUF, b)
        store_out(s, b)

    for b in range(_NBUF):
        start_gather(b, b)
    for b in range(_NBUF):
        step(b, b, wait_store=False, more_gathers=True)

    def loop_body(ss, _):
        s = ss * _NBUF
        for b in range(_NBUF):
            step(s + b, b, wait_store=True, more_gathers=True)
        return 0
    lax.fori_loop(1, _S // _NBUF - 1, loop_body, 0)

    for b in range(_NBUF):
        step(_S - _NBUF + b, b, wait_store=True, more_gathers=False)
    for b in range(_NBUF):
        wait_stores(0, b)


@jax.jit
def _embed(ids_t, W):
    mesh = plsc.VectorSubcoreMesh(core_axis_name="c", subcore_axis_name="s")
    return pl.kernel(
        _embed_kernel,
        mesh=mesh,
        out_type=jax.ShapeDtypeStruct((_S, 8, _TB, 8, 128), jnp.float32),
        scratch_types=[
            pltpu.VMEM((_S, _BW), jnp.int32),
            [pltpu.VMEM((_BW, 128), jnp.float32) for _ in range(_NBUF)],
            [pltpu.VMEM((_DIM, _PITCH), jnp.float32) for _ in range(_NBUF)],
            [pltpu.SemaphoreType.DMA for _ in range(_NBUF)],
            [pltpu.SemaphoreType.DMA for _ in range(_NBUF)],
        ],
        compiler_params=pltpu.CompilerParams(use_tc_tiling_on_sc=False,
                                             needs_layout_passes=False),
    )(ids_t, W)


def kernel(ids, W):
    Wp = jnp.pad(W, ((0, 0), (0, 128 - _DIM)))
    o5 = _embed(ids.astype(jnp.int32).T, Wp)
    return o5.transpose(2, 4, 0, 1, 3).reshape(_B, _S, _DIM)


# speedup vs baseline: 1.0219x; 1.0219x over previous
import math

import jax
import jax.numpy as jnp
from jax import lax
from jax.experimental import pallas as pl
from jax.experimental.pallas import tpu as pltpu
from jax.experimental.pallas import tpu_sc as plsc

_VOCAB = 1000000
_DIM = 64
_B = 4096
_S = 200
_NW = 32
_BW = _B // _NW
_TB = _B // 128
_SCALE = 1.0 / math.sqrt(_DIM)
_L = 16
_NBUF = 4
_PITCH = 129


def _embed_kernel(idst_hbm, table_hbm, out_hbm,
                  idx_v, gbufs, obufs, gsems, ssems):
    wid = lax.axis_index("s") * 2 + lax.axis_index("c")

    pltpu.sync_copy(idst_hbm.at[:, pl.ds(wid * _BW, _BW)], idx_v)

    iota = lax.iota(jnp.int32, _L)
    dvecs = [iota + c * _L for c in range(_DIM // _L)]

    def start_gather(s, b):
        pltpu.async_copy(table_hbm.at[idx_v.at[s]], gbufs[b], gsems[b])

    def transpose_scale(b):
        gbuf, obuf = gbufs[b], obufs[b]

        @plsc.parallel_loop(0, _BW, unroll=4)
        def _(bm):
            bmv = iota * 0 + bm
            for c in range(_DIM // _L):
                v = gbuf[bm, pl.ds(c * _L, _L)] * _SCALE
                plsc.store_scatter(obuf, [dvecs[c], bmv], v)

    def store_out(s, b):
        for td in range(8):
            pltpu.async_copy(
                obufs[b].at[pl.ds(td * 8, 8), pl.ds(0, 128)],
                out_hbm.at[s, td, wid], ssems[b])

    def wait_stores(s, b):
        for td in range(8):
            pltpu.make_async_copy(
                obufs[b].at[pl.ds(td * 8, 8), pl.ds(0, 128)],
                out_hbm.at[s, td, wid], ssems[b]).wait()

    def step(s, b, wait_store, more_gathers):
        pltpu.make_async_copy(table_hbm.at[idx_v.at[s]], gbufs[b],
                              gsems[b]).wait()
        if wait_store:
            wait_stores(s, b)
        transpose_scale(b)
        if more_gathers:
            start_gather(s + _NBUF, b)
        store_out(s, b)

    for b in range(_NBUF):
        start_gather(b, b)
    for b in range(_NBUF):
        step(b, b, wait_store=False, more_gathers=True)

    def loop_body(ss, _):
        s = ss * _NBUF
        for b in range(_NBUF):
            step(s + b, b, wait_store=True, more_gathers=True)
        return 0
    lax.fori_loop(1, _S // _NBUF - 1, loop_body, 0)

    for b in range(_NBUF):
        step(_S - _NBUF + b, b, wait_store=True, more_gathers=False)
    for b in range(_NBUF):
        wait_stores(0, b)


@jax.jit
def _embed(ids_t, W):
    mesh = plsc.VectorSubcoreMesh(core_axis_name="c", subcore_axis_name="s")
    return pl.kernel(
        _embed_kernel,
        mesh=mesh,
        out_type=jax.ShapeDtypeStruct((_S, 8, _TB, 8, 128), jnp.float32),
        scratch_types=[
            pltpu.VMEM((_S, _BW), jnp.int32),
            [pltpu.VMEM((_BW, 128), jnp.float32) for _ in range(_NBUF)],
            [pltpu.VMEM((_DIM, _PITCH), jnp.float32) for _ in range(_NBUF)],
            [pltpu.SemaphoreType.DMA for _ in range(_NBUF)],
            [pltpu.SemaphoreType.DMA for _ in range(_NBUF)],
        ],
        compiler_params=pltpu.CompilerParams(use_tc_tiling_on_sc=False,
                                             needs_layout_passes=False),
    )(ids_t, W)


def kernel(ids, W):
    Wp = jnp.pad(W, ((0, 0), (0, 128 - _DIM)))
    o5 = _embed(ids.astype(jnp.int32).T, Wp)
    return o5.transpose(2, 4, 0, 1, 3).reshape(_B, _S, _DIM)
